# baseline (speedup 1.0000x reference)
import functools

import jax
import jax.numpy as jnp
from jax import lax
from jax.experimental import pallas as pl
from jax.experimental.pallas import tpu as pltpu
from jax.experimental.pallas import tpu_sc as plsc

N_VAR, N_CON, E, D = 100000, 50000, 1600000, 16


def _node_body(x_ref, w1_ref, b1_ref, w2_ref, b2_ref, wp_ref, o_ref):
    h = jnp.maximum(jnp.dot(w1_ref[...], x_ref[...],
                            preferred_element_type=jnp.float32) + b1_ref[...], 0.0)
    h = jnp.maximum(jnp.dot(w2_ref[...], h,
                            preferred_element_type=jnp.float32) + b2_ref[...], 0.0)
    o_ref[...] = jnp.dot(wp_ref[...], h, preferred_element_type=jnp.float32)


def _node(xT, w1, b1, w2, b2, wp, block):
    n = xT.shape[1]
    block = min(block, n)
    wspec = pl.BlockSpec((16, 16), lambda i: (0, 0))
    bspec = pl.BlockSpec((16, 1), lambda i: (0, 0))
    return pl.pallas_call(
        _node_body,
        grid=(n // block,),
        in_specs=[pl.BlockSpec((16, block), lambda i: (0, i)),
                  wspec, bspec, wspec, bspec, wspec],
        out_specs=pl.BlockSpec((16, block), lambda i: (0, i)),
        out_shape=jax.ShapeDtypeStruct((16, n), jnp.float32),
    )(xT, w1.T, b1.reshape(16, 1), w2.T, b2.reshape(16, 1), wp.T)


def _node2_body(x_ref, vw1, vb1, vw2, vb2, vwp, cw1, cb1, cw2, cb2, cwp, o_ref):
    def mlp(w1, b1, w2, b2, wp):
        h = jnp.maximum(jnp.dot(w1[...], x_ref[...],
                                preferred_element_type=jnp.float32) + b1[...], 0.0)
        h = jnp.maximum(jnp.dot(w2[...], h,
                                preferred_element_type=jnp.float32) + b2[...], 0.0)
        return jnp.dot(wp[...], h, preferred_element_type=jnp.float32)

    n = x_ref.shape[1]
    col = jax.lax.broadcasted_iota(jnp.int32, (1, n), 1)
    o_ref[...] = jnp.where(col < N_VAR,
                           mlp(vw1, vb1, vw2, vb2, vwp),
                           mlp(cw1, cb1, cw2, cb2, cwp))


def _node2(xcat, vws, cws):
    n = xcat.shape[1]
    wspec = pl.BlockSpec((16, 16), lambda: (0, 0))
    bspec = pl.BlockSpec((16, 1), lambda: (0, 0))
    (vw1, vb1, vw2, vb2, vwp) = vws
    (cw1, cb1, cw2, cb2, cwp) = cws
    return pl.pallas_call(
        _node2_body,
        in_specs=[pl.BlockSpec((16, n), lambda: (0, 0))]
        + [wspec, bspec, wspec, bspec, wspec] * 2,
        out_specs=pl.BlockSpec((16, n), lambda: (0, 0)),
        out_shape=jax.ShapeDtypeStruct((16, n), jnp.float32),
    )(xcat,
      vw1.T, vb1.reshape(16, 1), vw2.T, vb2.reshape(16, 1), vwp.T,
      cw1.T, cb1.reshape(16, 1), cw2.T, cb2.reshape(16, 1), cwp.T)


_CHUNK = 1000


def _gather_add(table, ei, n, offset):
    info = plsc.get_sparse_core_info()
    nc, ns = info.num_cores, info.num_subcores
    nw = nc * ns
    epw = n // nw
    nchunk = epw // _CHUNK
    C = _CHUNK

    mesh = plsc.VectorSubcoreMesh(core_axis_name="c", subcore_axis_name="s")

    @functools.partial(
        pl.kernel,
        out_type=jax.ShapeDtypeStruct((n // 8, 128), jnp.float32),
        mesh=mesh,
        compiler_params=pltpu.CompilerParams(use_tc_tiling_on_sc=False),
        scratch_types=[
            pltpu.VMEM((C,), jnp.int32), pltpu.VMEM((C,), jnp.int32),
            pltpu.VMEM((C,), jnp.int32), pltpu.VMEM((C,), jnp.int32),
            pltpu.VMEM((C, 16), jnp.float32), pltpu.VMEM((C, 16), jnp.float32),
            pltpu.VMEM((C, 16), jnp.float32), pltpu.VMEM((C, 16), jnp.float32),
            pltpu.VMEM((C // 8, 128), jnp.float32), pltpu.VMEM((C // 8, 128), jnp.float32),
            pltpu.SemaphoreType.DMA, pltpu.SemaphoreType.DMA,
            pltpu.SemaphoreType.DMA, pltpu.SemaphoreType.DMA,
            pltpu.SemaphoreType.DMA, pltpu.SemaphoreType.DMA,
        ],
    )
    def gk(varp_hbm, ei_hbm, out_hbm,
           i0a, i0b, i1a, i1b, va, vb, ca, cb, oba, obb,
           gsa, gsb, isa, isb, osa, osb):
        wid = lax.axis_index("s") * nc + lax.axis_index("c")
        wbase = wid * epw
        ibase = offset + wbase
        slot_a = (i0a, i1a, va, ca, oba, gsa, isa, osa)
        slot_b = (i0b, i1b, vb, cb, obb, gsb, isb, osb)

        def idx_start(e, s):
            (i0s, i1s, _, _, _, _, iss, _) = s
            pltpu.async_copy(ei_hbm.at[0, pl.ds(ibase + e * C, C)], i0s, iss)
            pltpu.async_copy(ei_hbm.at[1, pl.ds(ibase + e * C, C)], i1s, iss)

        def idx_wait(s):
            (i0s, i1s, _, _, _, _, iss, _) = s
            pltpu.make_async_copy(ei_hbm.at[0, pl.ds(ibase, C)], i0s, iss).wait()
            pltpu.make_async_copy(ei_hbm.at[1, pl.ds(ibase, C)], i1s, iss).wait()

        def gather_start(s):
            (i0s, i1s, vs, cs, _, gss, _, _) = s
            pltpu.async_copy(varp_hbm.at[i0s], vs, gss)
            pltpu.async_copy(varp_hbm.at[i1s], cs, gss)

        def gather_wait(s):
            (i0s, i1s, vs, cs, _, gss, _, _) = s
            pltpu.make_async_copy(varp_hbm.at[i0s], vs, gss).wait()
            pltpu.make_async_copy(varp_hbm.at[i1s], cs, gss).wait()

        def out_wait(s):
            (_, _, _, _, obs, _, _, oss) = s
            pltpu.make_async_copy(
                obs, out_hbm.at[pl.ds(wbase // 8, C // 8), :], oss).wait()

        def half(e, s, n):
            (i0s, i1s, vs, cs, obs, gss, iss, oss) = s
            gather_wait(s)

            @pl.when(e + 1 < nchunk)
            def _():
                idx_wait(n)
                gather_start(n)

            @pl.when(e + 2 < nchunk)
            def _():
                idx_start(e + 2, s)

            @pl.when(e >= 2)
            def _():
                out_wait(s)

            def addrow(r, c2):
                obs[r // 8, pl.ds((r % 8) * 16, 16)] = vs[r, :] + cs[r, :]
                return c2
            lax.fori_loop(0, C, addrow, 0, unroll=8)
            pltpu.async_copy(
                obs, out_hbm.at[pl.ds((wbase + e * C) // 8, C // 8), :], oss)

        (i0s, i1s, _, _, _, _, _, _) = slot_a
        pltpu.sync_copy(ei_hbm.at[0, pl.ds(ibase, C)], i0s)
        pltpu.sync_copy(ei_hbm.at[1, pl.ds(ibase, C)], i1s)
        gather_start(slot_a)
        idx_start(1, slot_b)

        def pair(k, carry):
            half(2 * k, slot_a, slot_b)
            half(2 * k + 1, slot_b, slot_a)
            return carry

        lax.fori_loop(0, nchunk // 2, pair, 0)
        if nchunk % 2:
            half(nchunk - 1, slot_a, slot_b)
        out_wait(slot_a)
        out_wait(slot_b)

    return gk(table, ei)


def _cast_body(x_ref, o_ref):
    o_ref[...] = x_ref[...].astype(jnp.bfloat16)


def _cast_bf16(x, block=8000):
    n = x.shape[0]
    block = min(block, n)
    return pl.pallas_call(
        _cast_body,
        grid=(n // block,),
        in_specs=[pl.BlockSpec((block, 128), lambda i: (i, 0))],
        out_specs=pl.BlockSpec((block, 128), lambda i: (i, 0)),
        out_shape=jax.ShapeDtypeStruct((n, 128), jnp.bfloat16),
    )(x)


def _edge_body(ceT_ref, gT_ref, w1_ref, b1_ref, w2_ref, b2_ref, o_ref):
    pre = jnp.dot(w1_ref[...], ceT_ref[...],
                  preferred_element_type=jnp.float32) \
        + gT_ref[...].astype(jnp.float32) + b1_ref[...]
    h = jnp.maximum(pre, 0.0)
    o_ref[...] = jnp.dot(w2_ref[...], h,
                         preferred_element_type=jnp.float32) + b2_ref[...]


def _edge_alias_body(ceT_ref, gT_ref, w1_ref, b1_ref, w2_ref, b2_ref,
                     prev_ref, o_ref):
    _edge_body(ceT_ref, gT_ref, w1_ref, b1_ref, w2_ref, b2_ref, o_ref)


def _edge_half(ceT, gTh, w1e, b1, w2, b2, h, prev, block=32000):
    nh = gTh.shape[1]
    block = min(block, nh)
    nblk = nh // block
    off = h * nblk
    wspec = pl.BlockSpec((16, 16), lambda i: (0, 0))
    bspec = pl.BlockSpec((16, 1), lambda i: (0, 0))
    in_specs = [pl.BlockSpec((16, block), lambda i: (0, i + off)),
                pl.BlockSpec((16, block), lambda i: (0, i)),
                wspec, bspec, wspec, bspec]
    args = [ceT, gTh, w1e.T, b1.reshape(16, 1), w2.T, b2.reshape(16, 1)]
    kwargs = {}
    body = _edge_body
    if prev is not None:
        in_specs.append(pl.BlockSpec(memory_space=pl.ANY))
        args.append(prev)
        kwargs["input_output_aliases"] = {6: 0}
        body = _edge_alias_body
    return pl.pallas_call(
        body,
        grid=(nblk,),
        in_specs=in_specs,
        out_specs=pl.BlockSpec((16, block), lambda i: (0, i + off)),
        out_shape=jax.ShapeDtypeStruct((16, E), jnp.float32),
        **kwargs,
    )(*args)


def kernel(var_f, con_f, combined_edge_f, edge_index_var_con,
           vW1, vb1, vW2, vb2, cW1, cb1, cW2, cb2, eW1, eb1, eW2, eb2):
    eW1_e, eW1_v, eW1_c = eW1[:16], eW1[16:32], eW1[32:48]
    xcat = jnp.concatenate([var_f.T, con_f.T], axis=1)
    catT = _node2(xcat, (vW1, vb1, vW2, vb2, eW1_v), (cW1, cb1, cW2, cb2, eW1_c))
    table = catT.T
    ei2 = edge_index_var_con + jnp.array([[0], [N_VAR]], dtype=edge_index_var_con.dtype)
    ceT = combined_edge_f.T
    nh = E // 2
    out = None
    for h in range(2):
        g8 = _gather_add(table, ei2, nh, h * nh)
        gTh = _cast_bf16(g8).reshape(nh, 16).T
        out = _edge_half(ceT, gTh, eW1_e, eb1, eW2, eb2, h, out)
    return out.T


# speedup vs baseline: 1.0115x; 1.0115x over previous
import functools

import jax
import jax.numpy as jnp
from jax import lax
from jax.experimental import pallas as pl
from jax.experimental.pallas import tpu as pltpu
from jax.experimental.pallas import tpu_sc as plsc

N_VAR, N_CON, E, D = 100000, 50000, 1600000, 16


def _node_body(x_ref, w1_ref, b1_ref, w2_ref, b2_ref, wp_ref, o_ref):
    h = jnp.maximum(jnp.dot(w1_ref[...], x_ref[...],
                            preferred_element_type=jnp.float32) + b1_ref[...], 0.0)
    h = jnp.maximum(jnp.dot(w2_ref[...], h,
                            preferred_element_type=jnp.float32) + b2_ref[...], 0.0)
    o_ref[...] = jnp.dot(wp_ref[...], h, preferred_element_type=jnp.float32)


def _node(xT, w1, b1, w2, b2, wp, block):
    n = xT.shape[1]
    block = min(block, n)
    wspec = pl.BlockSpec((16, 16), lambda i: (0, 0))
    bspec = pl.BlockSpec((16, 1), lambda i: (0, 0))
    return pl.pallas_call(
        _node_body,
        grid=(n // block,),
        in_specs=[pl.BlockSpec((16, block), lambda i: (0, i)),
                  wspec, bspec, wspec, bspec, wspec],
        out_specs=pl.BlockSpec((16, block), lambda i: (0, i)),
        out_shape=jax.ShapeDtypeStruct((16, n), jnp.float32),
    )(xT, w1.T, b1.reshape(16, 1), w2.T, b2.reshape(16, 1), wp.T)


_CHUNK = 1000


def _gather_add(var_p, con_p, idx0, n, offset):
    info = plsc.get_sparse_core_info()
    nc, ns = info.num_cores, info.num_subcores
    nw = nc * ns
    epw = n // nw
    nchunk = epw // _CHUNK
    C = _CHUNK

    mesh = plsc.VectorSubcoreMesh(core_axis_name="c", subcore_axis_name="s")

    @functools.partial(
        pl.kernel,
        out_type=jax.ShapeDtypeStruct((n // 8, 128), jnp.float32),
        mesh=mesh,
        compiler_params=pltpu.CompilerParams(use_tc_tiling_on_sc=False),
        scratch_types=[
            pltpu.VMEM((C,), jnp.int32), pltpu.VMEM((C,), jnp.int32),
            pltpu.VMEM((C,), jnp.int32), pltpu.VMEM((C,), jnp.int32),
            pltpu.VMEM((C, 16), jnp.float32), pltpu.VMEM((C, 16), jnp.float32),
            pltpu.VMEM((C, 16), jnp.float32), pltpu.VMEM((C, 16), jnp.float32),
            pltpu.VMEM((C // 8, 128), jnp.float32), pltpu.VMEM((C // 8, 128), jnp.float32),
            pltpu.SemaphoreType.DMA, pltpu.SemaphoreType.DMA,
            pltpu.SemaphoreType.DMA, pltpu.SemaphoreType.DMA,
            pltpu.SemaphoreType.DMA, pltpu.SemaphoreType.DMA,
        ],
    )
    def gk(varp_hbm, conp_hbm, ei_hbm, out_hbm,
           i0a, i0b, i1a, i1b, va, vb, ca, cb, oba, obb,
           gsa, gsb, isa, isb, osa, osb):
        wid = lax.axis_index("s") * nc + lax.axis_index("c")
        wbase = wid * epw
        ibase = offset + wbase
        slot_a = (i0a, i1a, va, ca, oba, gsa, isa, osa)
        slot_b = (i0b, i1b, vb, cb, obb, gsb, isb, osb)

        def idx_start(e, s):
            (i0s, i1s, _, _, _, _, iss, _) = s
            pltpu.async_copy(ei_hbm.at[0, pl.ds(ibase + e * C, C)], i0s, iss)
            pltpu.async_copy(ei_hbm.at[1, pl.ds(ibase + e * C, C)], i1s, iss)

        def idx_wait(s):
            (i0s, i1s, _, _, _, _, iss, _) = s
            pltpu.make_async_copy(ei_hbm.at[0, pl.ds(ibase, C)], i0s, iss).wait()
            pltpu.make_async_copy(ei_hbm.at[1, pl.ds(ibase, C)], i1s, iss).wait()

        def gather_start(s):
            (i0s, i1s, vs, cs, _, gss, _, _) = s
            pltpu.async_copy(varp_hbm.at[i0s], vs, gss)
            pltpu.async_copy(conp_hbm.at[i1s], cs, gss)

        def gather_wait(s):
            (i0s, i1s, vs, cs, _, gss, _, _) = s
            pltpu.make_async_copy(varp_hbm.at[i0s], vs, gss).wait()
            pltpu.make_async_copy(conp_hbm.at[i1s], cs, gss).wait()

        def out_wait(s):
            (_, _, _, _, obs, _, _, oss) = s
            pltpu.make_async_copy(
                obs, out_hbm.at[pl.ds(wbase // 8, C // 8), :], oss).wait()

        def half(e, s, n):
            (i0s, i1s, vs, cs, obs, gss, iss, oss) = s
            gather_wait(s)

            @pl.when(e + 1 < nchunk)
            def _():
                idx_wait(n)
                gather_start(n)

            @pl.when(e + 2 < nchunk)
            def _():
                idx_start(e + 2, s)

            @pl.when(e >= 2)
            def _():
                out_wait(s)

            def addrow(r, c2):
                obs[r // 8, pl.ds((r % 8) * 16, 16)] = vs[r, :] + cs[r, :]
                return c2
            lax.fori_loop(0, C, addrow, 0, unroll=8)
            pltpu.async_copy(
                obs, out_hbm.at[pl.ds((wbase + e * C) // 8, C // 8), :], oss)

        (i0s, i1s, _, _, _, _, _, _) = slot_a
        pltpu.sync_copy(ei_hbm.at[0, pl.ds(ibase, C)], i0s)
        pltpu.sync_copy(ei_hbm.at[1, pl.ds(ibase, C)], i1s)
        gather_start(slot_a)
        idx_start(1, slot_b)

        def pair(k, carry):
            half(2 * k, slot_a, slot_b)
            half(2 * k + 1, slot_b, slot_a)
            return carry

        lax.fori_loop(0, nchunk // 2, pair, 0)
        if nchunk % 2:
            half(nchunk - 1, slot_a, slot_b)
        out_wait(slot_a)
        out_wait(slot_b)

    return gk(var_p, con_p, idx0)


def _cast_body(x_ref, o_ref):
    o_ref[...] = x_ref[...].astype(jnp.bfloat16)


def _cast_bf16(x, block=8000):
    n = x.shape[0]
    block = min(block, n)
    return pl.pallas_call(
        _cast_body,
        grid=(n // block,),
        in_specs=[pl.BlockSpec((block, 128), lambda i: (i, 0))],
        out_specs=pl.BlockSpec((block, 128), lambda i: (i, 0)),
        out_shape=jax.ShapeDtypeStruct((n, 128), jnp.bfloat16),
    )(x)


def _edge_body(ceT_ref, gT_ref, w1_ref, b1_ref, w2_ref, b2_ref, o_ref):
    pre = jnp.dot(w1_ref[...], ceT_ref[...],
                  preferred_element_type=jnp.float32) \
        + gT_ref[...].astype(jnp.float32) + b1_ref[...]
    h = jnp.maximum(pre, 0.0)
    o_ref[...] = jnp.dot(w2_ref[...], h,
                         preferred_element_type=jnp.float32) + b2_ref[...]


def _edge_alias_body(ceT_ref, gT_ref, w1_ref, b1_ref, w2_ref, b2_ref,
                     prev_ref, o_ref):
    _edge_body(ceT_ref, gT_ref, w1_ref, b1_ref, w2_ref, b2_ref, o_ref)


def _edge_half(ceT, gTh, w1e, b1, w2, b2, h, prev, block=32000):
    nh = gTh.shape[1]
    block = min(block, nh)
    nblk = nh // block
    off = h * nblk
    wspec = pl.BlockSpec((16, 16), lambda i: (0, 0))
    bspec = pl.BlockSpec((16, 1), lambda i: (0, 0))
    in_specs = [pl.BlockSpec((16, block), lambda i: (0, i + off)),
                pl.BlockSpec((16, block), lambda i: (0, i)),
                wspec, bspec, wspec, bspec]
    args = [ceT, gTh, w1e.T, b1.reshape(16, 1), w2.T, b2.reshape(16, 1)]
    kwargs = {}
    body = _edge_body
    if prev is not None:
        in_specs.append(pl.BlockSpec(memory_space=pl.ANY))
        args.append(prev)
        kwargs["input_output_aliases"] = {6: 0}
        body = _edge_alias_body
    return pl.pallas_call(
        body,
        grid=(nblk,),
        in_specs=in_specs,
        out_specs=pl.BlockSpec((16, block), lambda i: (0, i + off)),
        out_shape=jax.ShapeDtypeStruct((16, E), jnp.float32),
        **kwargs,
    )(*args)


def kernel(var_f, con_f, combined_edge_f, edge_index_var_con,
           vW1, vb1, vW2, vb2, cW1, cb1, cW2, cb2, eW1, eb1, eW2, eb2):
    eW1_e, eW1_v, eW1_c = eW1[:16], eW1[16:32], eW1[32:48]
    var_pT = _node(var_f.T, vW1, vb1, vW2, vb2, eW1_v, block=N_VAR)
    con_pT = _node(con_f.T, cW1, cb1, cW2, cb2, eW1_c, block=N_CON)
    var_p, con_p = var_pT.T, con_pT.T
    ceT = combined_edge_f.T
    nh = E // 2
    out = None
    for h in range(2):
        g8 = _gather_add(var_p, con_p, edge_index_var_con, nh, h * nh)
        gTh = g8.astype(jnp.bfloat16).reshape(nh, 16).T
        out = _edge_half(ceT, gTh, eW1_e, eb1, eW2, eb2, h, out)
    return out.T


# speedup vs baseline: 1.0164x; 1.0048x over previous
import functools

import jax
import jax.numpy as jnp
from jax import lax
from jax.experimental import pallas as pl
from jax.experimental.pallas import tpu as pltpu
from jax.experimental.pallas import tpu_sc as plsc

N_VAR, N_CON, E, D = 100000, 50000, 1600000, 16


def _node_body(x_ref, w1_ref, b1_ref, w2_ref, b2_ref, wp_ref, o_ref):
    h = jnp.maximum(jnp.dot(w1_ref[...], x_ref[...],
                            preferred_element_type=jnp.float32) + b1_ref[...], 0.0)
    h = jnp.maximum(jnp.dot(w2_ref[...], h,
                            preferred_element_type=jnp.float32) + b2_ref[...], 0.0)
    o_ref[...] = jnp.dot(wp_ref[...], h, preferred_element_type=jnp.float32)


def _node(xT, w1, b1, w2, b2, wp, block):
    n = xT.shape[1]
    block = min(block, n)
    wspec = pl.BlockSpec((16, 16), lambda i: (0, 0))
    bspec = pl.BlockSpec((16, 1), lambda i: (0, 0))
    return pl.pallas_call(
        _node_body,
        grid=(n // block,),
        in_specs=[pl.BlockSpec((16, block), lambda i: (0, i)),
                  wspec, bspec, wspec, bspec, wspec],
        out_specs=pl.BlockSpec((16, block), lambda i: (0, i)),
        out_shape=jax.ShapeDtypeStruct((16, n), jnp.float32),
    )(xT, w1.T, b1.reshape(16, 1), w2.T, b2.reshape(16, 1), wp.T)


_CHUNK = 1000


def _gather_add(var_p, con_p, idx0, n, offset):
    info = plsc.get_sparse_core_info()
    nc, ns = info.num_cores, info.num_subcores
    nw = nc * ns
    epw = n // nw
    nchunk = epw // _CHUNK
    C = _CHUNK

    mesh = plsc.VectorSubcoreMesh(core_axis_name="c", subcore_axis_name="s")

    @functools.partial(
        pl.kernel,
        out_type=jax.ShapeDtypeStruct((n // 8, 128), jnp.float32),
        mesh=mesh,
        compiler_params=pltpu.CompilerParams(use_tc_tiling_on_sc=False),
        scratch_types=[
            pltpu.VMEM((C,), jnp.int32), pltpu.VMEM((C,), jnp.int32),
            pltpu.VMEM((C,), jnp.int32), pltpu.VMEM((C,), jnp.int32),
            pltpu.VMEM((C, 16), jnp.float32), pltpu.VMEM((C, 16), jnp.float32),
            pltpu.VMEM((C, 16), jnp.float32), pltpu.VMEM((C, 16), jnp.float32),
            pltpu.VMEM((C // 8, 128), jnp.float32), pltpu.VMEM((C // 8, 128), jnp.float32),
            pltpu.SemaphoreType.DMA, pltpu.SemaphoreType.DMA,
            pltpu.SemaphoreType.DMA, pltpu.SemaphoreType.DMA,
            pltpu.SemaphoreType.DMA, pltpu.SemaphoreType.DMA,
        ],
    )
    def gk(varp_hbm, conp_hbm, ei_hbm, out_hbm,
           i0a, i0b, i1a, i1b, va, vb, ca, cb, oba, obb,
           gsa, gsb, isa, isb, osa, osb):
        wid = lax.axis_index("s") * nc + lax.axis_index("c")
        wbase = wid * epw
        ibase = offset + wbase
        slot_a = (i0a, i1a, va, ca, oba, gsa, isa, osa)
        slot_b = (i0b, i1b, vb, cb, obb, gsb, isb, osb)

        def idx_start(e, s):
            (i0s, i1s, _, _, _, _, iss, _) = s
            pltpu.async_copy(ei_hbm.at[0, pl.ds(ibase + e * C, C)], i0s, iss)
            pltpu.async_copy(ei_hbm.at[1, pl.ds(ibase + e * C, C)], i1s, iss)

        def idx_wait(s):
            (i0s, i1s, _, _, _, _, iss, _) = s
            pltpu.make_async_copy(ei_hbm.at[0, pl.ds(ibase, C)], i0s, iss).wait()
            pltpu.make_async_copy(ei_hbm.at[1, pl.ds(ibase, C)], i1s, iss).wait()

        def gather_start(s):
            (i0s, i1s, vs, cs, _, gss, _, _) = s
            pltpu.async_copy(varp_hbm.at[i0s], vs, gss)
            pltpu.async_copy(conp_hbm.at[i1s], cs, gss)

        def gather_wait(s):
            (i0s, i1s, vs, cs, _, gss, _, _) = s
            pltpu.make_async_copy(varp_hbm.at[i0s], vs, gss).wait()
            pltpu.make_async_copy(conp_hbm.at[i1s], cs, gss).wait()

        def out_wait(s):
            (_, _, _, _, obs, _, _, oss) = s
            pltpu.make_async_copy(
                obs, out_hbm.at[pl.ds(wbase // 8, C // 8), :], oss).wait()

        def half(e, s, n):
            (i0s, i1s, vs, cs, obs, gss, iss, oss) = s
            gather_wait(s)

            @pl.when(e + 1 < nchunk)
            def _():
                idx_wait(n)
                gather_start(n)

            @pl.when(e + 2 < nchunk)
            def _():
                idx_start(e + 2, s)

            @pl.when(e >= 2)
            def _():
                out_wait(s)

            def addrow(r, c2):
                obs[r // 8, pl.ds((r % 8) * 16, 16)] = vs[r, :] + cs[r, :]
                return c2
            lax.fori_loop(0, C, addrow, 0, unroll=8)
            pltpu.async_copy(
                obs, out_hbm.at[pl.ds((wbase + e * C) // 8, C // 8), :], oss)

        (i0s, i1s, _, _, _, _, _, _) = slot_a
        pltpu.sync_copy(ei_hbm.at[0, pl.ds(ibase, C)], i0s)
        pltpu.sync_copy(ei_hbm.at[1, pl.ds(ibase, C)], i1s)
        gather_start(slot_a)
        idx_start(1, slot_b)

        def pair(k, carry):
            half(2 * k, slot_a, slot_b)
            half(2 * k + 1, slot_b, slot_a)
            return carry

        lax.fori_loop(0, nchunk // 2, pair, 0)
        if nchunk % 2:
            half(nchunk - 1, slot_a, slot_b)
        out_wait(slot_a)
        out_wait(slot_b)

    return gk(var_p, con_p, idx0)


def _cast_body(x_ref, o_ref):
    o_ref[...] = x_ref[...].astype(jnp.bfloat16)


def _cast_bf16(x, block=8000):
    n = x.shape[0]
    block = min(block, n)
    return pl.pallas_call(
        _cast_body,
        grid=(n // block,),
        in_specs=[pl.BlockSpec((block, 128), lambda i: (i, 0))],
        out_specs=pl.BlockSpec((block, 128), lambda i: (i, 0)),
        out_shape=jax.ShapeDtypeStruct((n, 128), jnp.bfloat16),
    )(x)


def _edge_body(ceT_ref, gT_ref, w1_ref, b1_ref, w2_ref, b2_ref, o_ref):
    pre = jnp.dot(w1_ref[...], ceT_ref[...],
                  preferred_element_type=jnp.float32) \
        + gT_ref[...].astype(jnp.float32) + b1_ref[...]
    h = jnp.maximum(pre, 0.0)
    o_ref[...] = jnp.dot(w2_ref[...], h,
                         preferred_element_type=jnp.float32) + b2_ref[...]


def _edge_alias_body(ceT_ref, gT_ref, w1_ref, b1_ref, w2_ref, b2_ref,
                     prev_ref, o_ref):
    _edge_body(ceT_ref, gT_ref, w1_ref, b1_ref, w2_ref, b2_ref, o_ref)


def _edge_half(ceT, gTh, w1e, b1, w2, b2, h, prev, block=32000):
    nh = gTh.shape[1]
    block = min(block, nh)
    nblk = nh // block
    off = h * nblk
    wspec = pl.BlockSpec((16, 16), lambda i: (0, 0))
    bspec = pl.BlockSpec((16, 1), lambda i: (0, 0))
    in_specs = [pl.BlockSpec((16, block), lambda i: (0, i + off)),
                pl.BlockSpec((16, block), lambda i: (0, i)),
                wspec, bspec, wspec, bspec]
    args = [ceT, gTh, w1e.T, b1.reshape(16, 1), w2.T, b2.reshape(16, 1)]
    kwargs = {}
    body = _edge_body
    if prev is not None:
        in_specs.append(pl.BlockSpec(memory_space=pl.ANY))
        args.append(prev)
        kwargs["input_output_aliases"] = {6: 0}
        body = _edge_alias_body
    return pl.pallas_call(
        body,
        grid=(nblk,),
        in_specs=in_specs,
        out_specs=pl.BlockSpec((16, block), lambda i: (0, i + off)),
        out_shape=jax.ShapeDtypeStruct((16, E), jnp.float32),
        **kwargs,
    )(*args)


def kernel(var_f, con_f, combined_edge_f, edge_index_var_con,
           vW1, vb1, vW2, vb2, cW1, cb1, cW2, cb2, eW1, eb1, eW2, eb2):
    eW1_e, eW1_v, eW1_c = eW1[:16], eW1[16:32], eW1[32:48]
    var_pT = _node(var_f.T, vW1, vb1, vW2, vb2, eW1_v, block=N_VAR)
    con_pT = _node(con_f.T, cW1, cb1, cW2, cb2, eW1_c, block=N_CON)
    var_p, con_p = var_pT.T, con_pT.T
    ceT = combined_edge_f.T
    nh = E // 2
    out = None
    for h in range(2):
        g8 = _gather_add(var_p, con_p, edge_index_var_con, nh, h * nh)
        gTh = _cast_bf16(g8).reshape(nh, 16).T
        out = _edge_half(ceT, gTh, eW1_e, eb1, eW2, eb2, h, out)
    return out.T
